# baseline (speedup 1.0000x reference)
import functools

import jax
import jax.numpy as jnp
from jax import lax
from jax.experimental import pallas as pl
from jax.experimental.pallas import tpu as pltpu
from jax.experimental.pallas import tpu_sc as plsc

N = 10000
M = 5000
E = 320000
D = 128
F = 8

GROUP = 128
NTILES = 32
GP_T = 80
NGROUPS = NTILES * GP_T
E_PAD = NGROUPS * GROUP
NBUF = 8
ROUNDS = GP_T // NBUF
M_PAD = 5120
N_PAD = 10240
ZROWS = N_PAD // 16

ROWS_BLK = 2000


def _sc_stage(acc_rows):
    zchunks = acc_rows // ZROWS
    mesh = plsc.VectorSubcoreMesh(core_axis_name="c", subcore_axis_name="s")

    def body(table_hbm, gidx_hbm, sidx_hbm, z_hbm, out_hbm,
             gidxv, sidxv, rows, acc, *gsems):
        cid = lax.axis_index("c")
        sid = lax.axis_index("s")
        wid = cid * 16 + sid
        gbase = wid * GP_T

        pltpu.sync_copy(gidx_hbm.at[pl.ds(gbase, GP_T)], gidxv)
        pltpu.sync_copy(sidx_hbm.at[pl.ds(gbase, GP_T)], sidxv)
        for zc in range(zchunks):
            pltpu.sync_copy(z_hbm, acc.at[pl.ds(zc * ZROWS, ZROWS)])

        for b in range(NBUF):
            pltpu.async_copy(table_hbm.at[gidxv.at[b]], rows.at[b], gsems[b])

        def round_body(i, carry):
            iota16 = lax.iota(jnp.int32, 16)
            col8 = iota16 % F
            rep2 = iota16 // F
            for b in range(NBUF):
                g = i * NBUF + b
                sidx_row = sidxv.at[g]
                pltpu.make_async_copy(
                    table_hbm.at[gidxv.at[b]], rows.at[b], gsems[b]).wait()
                rbuf = rows.at[b]
                for j in range(GROUP // 2):
                    pat = rep2 + (2 * j)
                    seg = plsc.load_gather(sidx_row, [pat])
                    val = plsc.load_gather(rbuf, [pat, col8])
                    plsc.addupdate_scatter(acc, [seg, col8], val)

                @pl.when(i < ROUNDS - 1)
                def _():
                    pltpu.async_copy(
                        table_hbm.at[gidxv.at[g + NBUF]], rows.at[b], gsems[b])
            return carry

        lax.fori_loop(0, ROUNDS, round_body, 0)
        pltpu.sync_copy(acc, out_hbm.at[wid])

    return pl.kernel(
        body,
        out_type=jax.ShapeDtypeStruct((NTILES, acc_rows, F), jnp.float32),
        mesh=mesh,
        compiler_params=pltpu.CompilerParams(
            use_tc_tiling_on_sc=False, needs_layout_passes=False),
        scratch_types=[
            pltpu.VMEM((GP_T, GROUP), jnp.int32),
            pltpu.VMEM((GP_T, GROUP), jnp.int32),
            pltpu.VMEM((NBUF, GROUP, F), jnp.float32),
            pltpu.VMEM((acc_rows, F), jnp.float32),
        ] + [pltpu.SemaphoreType.DMA] * NBUF,
    )


_s1 = _sc_stage(M_PAD)
_s2 = _sc_stage(N_PAD)


def _ka_body(x_ref, wm_ref, wu_ref, c_ref, b_ref, m8_ref, u8_ref):
    xb = x_ref[...]
    m8_ref[...] = jax.nn.gelu(
        jnp.dot(xb, wm_ref[...], preferred_element_type=jnp.float32)) + c_ref[...]
    u8_ref[...] = jnp.dot(
        xb, wu_ref[...], preferred_element_type=jnp.float32) + b_ref[...]


def _ka(x, wm8, wu8, c8, b8):
    return pl.pallas_call(
        _ka_body,
        grid=(N // ROWS_BLK,),
        in_specs=[
            pl.BlockSpec((ROWS_BLK, D), lambda i: (i, 0)),
            pl.BlockSpec((D, F), lambda i: (0, 0)),
            pl.BlockSpec((D, F), lambda i: (0, 0)),
            pl.BlockSpec((1, F), lambda i: (0, 0)),
            pl.BlockSpec((1, F), lambda i: (0, 0)),
        ],
        out_specs=[
            pl.BlockSpec((ROWS_BLK, F), lambda i: (i, 0)),
            pl.BlockSpec((ROWS_BLK, F), lambda i: (i, 0)),
        ],
        out_shape=[
            jax.ShapeDtypeStruct((N, F), jnp.float32),
            jax.ShapeDtypeStruct((N, F), jnp.float32),
        ],
    )(x, wm8, wu8, c8, b8)


def _ke_body(ep_ref, out_ref):
    p = jnp.sum(ep_ref[...], axis=0)
    cnt = jnp.maximum(p[:, 4:5], 1.0)
    out_ref[...] = p / cnt


KE_BLK = 640


def _ke(e_part):
    return pl.pallas_call(
        _ke_body,
        grid=(M_PAD // KE_BLK,),
        in_specs=[pl.BlockSpec((NTILES, KE_BLK, F), lambda i: (0, i, 0))],
        out_specs=pl.BlockSpec((KE_BLK, F), lambda i: (i, 0)),
        out_shape=jax.ShapeDtypeStruct((M_PAD, F), jnp.float32),
    )(e_part)


def _kb_body(vp_ref, u8_ref, out_ref):
    p = jnp.sum(vp_ref[...], axis=0)
    cnt = jnp.maximum(p[:, 4:5], 1.0)
    m_i = p[:, 0:4] / cnt
    h = jax.nn.gelu(u8_ref[:, 0:4] + m_i)
    hm = jnp.max(h, axis=1, keepdims=True)
    out_ref[...] = (h - hm) - jnp.log(
        jnp.sum(jnp.exp(h - hm), axis=1, keepdims=True))


KB_BLK = 1000


def _kb(v_part, u8):
    return pl.pallas_call(
        _kb_body,
        grid=(N // KB_BLK,),
        in_specs=[
            pl.BlockSpec((NTILES, KB_BLK, F), lambda i: (0, i, 0)),
            pl.BlockSpec((KB_BLK, F), lambda i: (i, 0)),
        ],
        out_specs=pl.BlockSpec((KB_BLK, 4), lambda i: (i, 0)),
        out_shape=jax.ShapeDtypeStruct((N, 4), jnp.float32),
    )(v_part, u8)


def kernel(x, v_idx, e_idx, W_msg, W_upd, b_upd):
    v_idx = v_idx.astype(jnp.int32)
    e_idx = e_idx.astype(jnp.int32)
    wm8 = jnp.pad(W_msg, ((0, 0), (0, F - 4)))
    wu8 = jnp.pad(W_upd, ((0, 0), (0, F - 4)))
    c8 = jnp.array([[0, 0, 0, 0, 1, 0, 0, 0]], jnp.float32)
    b8 = jnp.pad(b_upd, (0, F - 4)).reshape(1, F)
    z = jnp.zeros((ZROWS, F), jnp.float32)

    pad = E_PAD - E
    gs1 = jnp.pad(v_idx, (0, pad)).reshape(NGROUPS, GROUP)
    ss1 = jnp.pad(e_idx, (0, pad), constant_values=M).reshape(NGROUPS, GROUP)
    gs2 = jnp.pad(e_idx, (0, pad)).reshape(NGROUPS, GROUP)
    ss2 = jnp.pad(v_idx, (0, pad), constant_values=N).reshape(NGROUPS, GROUP)

    m8, u8 = _ka(x, wm8, wu8, c8, b8)
    e_part = _s1(m8, gs1, ss1, z)
    e_feat8 = _ke(e_part)
    v_part = _s2(e_feat8, gs2, ss2, z)
    return _kb(v_part, u8)


# speedup vs baseline: 2.5598x; 2.5598x over previous
import functools

import jax
import jax.numpy as jnp
from jax import lax
from jax.experimental import pallas as pl
from jax.experimental.pallas import tpu as pltpu
from jax.experimental.pallas import tpu_sc as plsc

N = 10000
M = 5000
E = 320000
D = 128
F = 8

GROUP = 128
NTILES = 32
GP_T = 80
NGROUPS = NTILES * GP_T
E_PAD = NGROUPS * GROUP
NBUF = 8
ROUNDS = GP_T // NBUF
M_PAD = 5120
N_PAD = 10240
ZROWS = N_PAD // 16

ROWS_BLK = 2000
KE_BLK = 640
KB_BLK = 1000


def _sc_stage(acc_rows):
    rpt = acc_rows // 16
    mesh = plsc.VectorSubcoreMesh(core_axis_name="c", subcore_axis_name="s")

    def body(table_hbm, gidx_hbm, sidx_hbm, z_hbm, out_hbm,
             gidxv, sidxv, rows, acc, *sems):
        cid = lax.axis_index("c")
        sid = lax.axis_index("s")
        wid = cid * 16 + sid
        gbase = wid * GP_T

        pltpu.sync_copy(gidx_hbm.at[pl.ds(gbase, GP_T)], gidxv)
        pltpu.sync_copy(sidx_hbm.at[pl.ds(gbase, GP_T)], sidxv)
        pltpu.sync_copy(z_hbm.at[pl.ds(0, rpt)], acc.at[pl.ds(sid * rpt, rpt)])

        for b in range(NBUF):
            pltpu.async_copy(table_hbm.at[gidxv.at[b]], rows.at[b], sems[b])
        plsc.subcore_barrier()

        def round_body(i, carry):
            for b in range(NBUF):
                g = i * NBUF + b
                pltpu.make_async_copy(
                    table_hbm.at[gidxv.at[b]], rows.at[b], sems[b]).wait()
                pltpu.sync_copy(rows.at[b], acc.at[sidxv.at[g]], add=True)

                @pl.when(i < ROUNDS - 1)
                def _():
                    pltpu.async_copy(
                        table_hbm.at[gidxv.at[g + NBUF]], rows.at[b], sems[b])
            return carry

        lax.fori_loop(0, ROUNDS, round_body, 0)
        plsc.subcore_barrier()
        pltpu.sync_copy(acc.at[pl.ds(sid * rpt, rpt)],
                        out_hbm.at[cid, pl.ds(sid * rpt, rpt)])

    return pl.kernel(
        body,
        out_type=jax.ShapeDtypeStruct((2, acc_rows, F), jnp.float32),
        mesh=mesh,
        compiler_params=pltpu.CompilerParams(use_tc_tiling_on_sc=False),
        scratch_types=[
            pltpu.VMEM((GP_T, GROUP), jnp.int32),
            pltpu.VMEM((GP_T, GROUP), jnp.int32),
            pltpu.VMEM((NBUF, GROUP, F), jnp.float32),
            pltpu.VMEM_SHARED((acc_rows, F), jnp.float32),
        ] + [pltpu.SemaphoreType.DMA] * NBUF,
    )


_s1 = _sc_stage(M_PAD)
_s2 = _sc_stage(N_PAD)


def _ka_body(x_ref, wm_ref, wu_ref, c_ref, b_ref, m8_ref, u8_ref):
    xb = x_ref[...]
    m8_ref[...] = jax.nn.gelu(
        jnp.dot(xb, wm_ref[...], preferred_element_type=jnp.float32)) + c_ref[...]
    u8_ref[...] = jnp.dot(
        xb, wu_ref[...], preferred_element_type=jnp.float32) + b_ref[...]


def _ka(x, wm8, wu8, c8, b8):
    return pl.pallas_call(
        _ka_body,
        grid=(N // ROWS_BLK,),
        in_specs=[
            pl.BlockSpec((ROWS_BLK, D), lambda i: (i, 0)),
            pl.BlockSpec((D, F), lambda i: (0, 0)),
            pl.BlockSpec((D, F), lambda i: (0, 0)),
            pl.BlockSpec((1, F), lambda i: (0, 0)),
            pl.BlockSpec((1, F), lambda i: (0, 0)),
        ],
        out_specs=[
            pl.BlockSpec((ROWS_BLK, F), lambda i: (i, 0)),
            pl.BlockSpec((ROWS_BLK, F), lambda i: (i, 0)),
        ],
        out_shape=[
            jax.ShapeDtypeStruct((N, F), jnp.float32),
            jax.ShapeDtypeStruct((N, F), jnp.float32),
        ],
    )(x, wm8, wu8, c8, b8)


def _ke_body(ep_ref, out_ref):
    p = ep_ref[0] + ep_ref[1]
    cnt = jnp.maximum(p[:, 4:5], 1.0)
    out_ref[...] = p / cnt


def _ke(e_part):
    return pl.pallas_call(
        _ke_body,
        grid=(M_PAD // KE_BLK,),
        in_specs=[pl.BlockSpec((2, KE_BLK, F), lambda i: (0, i, 0))],
        out_specs=pl.BlockSpec((KE_BLK, F), lambda i: (i, 0)),
        out_shape=jax.ShapeDtypeStruct((M_PAD, F), jnp.float32),
    )(e_part)


def _kb_body(vp_ref, u8_ref, out_ref):
    p = vp_ref[0] + vp_ref[1]
    cnt = jnp.maximum(p[:, 4:5], 1.0)
    m_i = p[:, 0:4] / cnt
    h = jax.nn.gelu(u8_ref[:, 0:4] + m_i)
    hm = jnp.max(h, axis=1, keepdims=True)
    out_ref[...] = (h - hm) - jnp.log(
        jnp.sum(jnp.exp(h - hm), axis=1, keepdims=True))


def _kb(v_part, u8):
    return pl.pallas_call(
        _kb_body,
        grid=(N // KB_BLK,),
        in_specs=[
            pl.BlockSpec((2, KB_BLK, F), lambda i: (0, i, 0)),
            pl.BlockSpec((KB_BLK, F), lambda i: (i, 0)),
        ],
        out_specs=pl.BlockSpec((KB_BLK, 4), lambda i: (i, 0)),
        out_shape=jax.ShapeDtypeStruct((N, 4), jnp.float32),
    )(v_part, u8)


def kernel(x, v_idx, e_idx, W_msg, W_upd, b_upd):
    v_idx = v_idx.astype(jnp.int32)
    e_idx = e_idx.astype(jnp.int32)
    wm8 = jnp.pad(W_msg, ((0, 0), (0, F - 4)))
    wu8 = jnp.pad(W_upd, ((0, 0), (0, F - 4)))
    c8 = jnp.array([[0, 0, 0, 0, 1, 0, 0, 0]], jnp.float32)
    b8 = jnp.pad(b_upd, (0, F - 4)).reshape(1, F)
    z = jnp.zeros((ZROWS, F), jnp.float32)

    pad = E_PAD - E
    gs1 = jnp.pad(v_idx, (0, pad)).reshape(NGROUPS, GROUP)
    ss1 = jnp.pad(e_idx, (0, pad), constant_values=M).reshape(NGROUPS, GROUP)
    gs2 = jnp.pad(e_idx, (0, pad)).reshape(NGROUPS, GROUP)
    ss2 = jnp.pad(v_idx, (0, pad), constant_values=N).reshape(NGROUPS, GROUP)

    m8, u8 = _ka(x, wm8, wu8, c8, b8)
    e_part = _s1(m8, gs1, ss1, z)
    e_feat8 = _ke(e_part)
    v_part = _s2(e_feat8, gs2, ss2, z)
    return _kb(v_part, u8)


# speedup vs baseline: 2.5604x; 1.0002x over previous
import functools

import jax
import jax.numpy as jnp
from jax import lax
from jax.experimental import pallas as pl
from jax.experimental.pallas import tpu as pltpu
from jax.experimental.pallas import tpu_sc as plsc

N = 10000
M = 5000
E = 320000
D = 128
F = 8

GROUP = 128
NTILES = 32
GP_T = 80
NGROUPS = NTILES * GP_T
E_PAD = NGROUPS * GROUP
NBUF = 8
ROUNDS = GP_T // NBUF
M_PAD = 5120
N_PAD = 10240
ZROWS = N_PAD // 16

ROWS_BLK = 2000
KE_BLK = 640
KB_BLK = 1000


def _sc_stage(acc_rows):
    rpt = acc_rows // 16
    mesh = plsc.VectorSubcoreMesh(core_axis_name="c", subcore_axis_name="s")

    def body(table_hbm, gidx_hbm, sidx_hbm, z_hbm, out_hbm,
             gidxv, sidxv, rows, acc, *sems):
        gsems = sems[:NBUF]
        ssems = sems[NBUF:]
        cid = lax.axis_index("c")
        sid = lax.axis_index("s")
        wid = cid * 16 + sid
        gbase = wid * GP_T

        pltpu.sync_copy(gidx_hbm.at[pl.ds(gbase, GP_T)], gidxv)
        pltpu.sync_copy(sidx_hbm.at[pl.ds(gbase, GP_T)], sidxv)
        pltpu.sync_copy(z_hbm.at[pl.ds(0, rpt)], acc.at[pl.ds(sid * rpt, rpt)])

        for b in range(NBUF):
            pltpu.async_copy(table_hbm.at[gidxv.at[b]], rows.at[b], gsems[b])
        plsc.subcore_barrier()

        def round_body(i, carry):
            for b in range(NBUF):
                g = i * NBUF + b
                pb = (b - 1) % NBUF
                pltpu.make_async_copy(
                    table_hbm.at[gidxv.at[b]], rows.at[b], gsems[b]).wait()
                pltpu.async_copy(rows.at[b], acc.at[sidxv.at[g]], ssems[b],
                                 add=True)

                @pl.when(jnp.logical_and(g >= 1, g <= GP_T - NBUF))
                def _():
                    pltpu.make_async_copy(
                        z_hbm.at[pl.ds(0, GROUP)], rows.at[pb],
                        ssems[pb]).wait()
                    pltpu.async_copy(
                        table_hbm.at[gidxv.at[g - 1 + NBUF]], rows.at[pb],
                        gsems[pb])
            return carry

        lax.fori_loop(0, ROUNDS, round_body, 0)
        for b in range(NBUF):
            pltpu.make_async_copy(
                z_hbm.at[pl.ds(0, GROUP)], rows.at[b], ssems[b]).wait()
        plsc.subcore_barrier()
        pltpu.sync_copy(acc.at[pl.ds(sid * rpt, rpt)],
                        out_hbm.at[cid, pl.ds(sid * rpt, rpt)])

    return pl.kernel(
        body,
        out_type=jax.ShapeDtypeStruct((2, acc_rows, F), jnp.float32),
        mesh=mesh,
        compiler_params=pltpu.CompilerParams(use_tc_tiling_on_sc=False),
        scratch_types=[
            pltpu.VMEM((GP_T, GROUP), jnp.int32),
            pltpu.VMEM((GP_T, GROUP), jnp.int32),
            pltpu.VMEM((NBUF, GROUP, F), jnp.float32),
            pltpu.VMEM_SHARED((acc_rows, F), jnp.float32),
        ] + [pltpu.SemaphoreType.DMA] * (2 * NBUF),
    )


_s1 = _sc_stage(M_PAD)
_s2 = _sc_stage(N_PAD)


def _ka_body(x_ref, wm_ref, wu_ref, c_ref, b_ref, m8_ref, u8_ref):
    xb = x_ref[...]
    m8_ref[...] = jax.nn.gelu(
        jnp.dot(xb, wm_ref[...], preferred_element_type=jnp.float32)) + c_ref[...]
    u8_ref[...] = jnp.dot(
        xb, wu_ref[...], preferred_element_type=jnp.float32) + b_ref[...]


def _ka(x, wm8, wu8, c8, b8):
    return pl.pallas_call(
        _ka_body,
        grid=(N // ROWS_BLK,),
        in_specs=[
            pl.BlockSpec((ROWS_BLK, D), lambda i: (i, 0)),
            pl.BlockSpec((D, F), lambda i: (0, 0)),
            pl.BlockSpec((D, F), lambda i: (0, 0)),
            pl.BlockSpec((1, F), lambda i: (0, 0)),
            pl.BlockSpec((1, F), lambda i: (0, 0)),
        ],
        out_specs=[
            pl.BlockSpec((ROWS_BLK, F), lambda i: (i, 0)),
            pl.BlockSpec((ROWS_BLK, F), lambda i: (i, 0)),
        ],
        out_shape=[
            jax.ShapeDtypeStruct((N, F), jnp.float32),
            jax.ShapeDtypeStruct((N, F), jnp.float32),
        ],
    )(x, wm8, wu8, c8, b8)


def _ke_body(ep_ref, out_ref):
    p = ep_ref[0] + ep_ref[1]
    cnt = jnp.maximum(p[:, 4:5], 1.0)
    out_ref[...] = p / cnt


def _ke(e_part):
    return pl.pallas_call(
        _ke_body,
        grid=(M_PAD // KE_BLK,),
        in_specs=[pl.BlockSpec((2, KE_BLK, F), lambda i: (0, i, 0))],
        out_specs=pl.BlockSpec((KE_BLK, F), lambda i: (i, 0)),
        out_shape=jax.ShapeDtypeStruct((M_PAD, F), jnp.float32),
    )(e_part)


def _kb_body(vp_ref, u8_ref, out_ref):
    p = vp_ref[0] + vp_ref[1]
    cnt = jnp.maximum(p[:, 4:5], 1.0)
    m_i = p[:, 0:4] / cnt
    h = jax.nn.gelu(u8_ref[:, 0:4] + m_i)
    hm = jnp.max(h, axis=1, keepdims=True)
    out_ref[...] = (h - hm) - jnp.log(
        jnp.sum(jnp.exp(h - hm), axis=1, keepdims=True))


def _kb(v_part, u8):
    return pl.pallas_call(
        _kb_body,
        grid=(N // KB_BLK,),
        in_specs=[
            pl.BlockSpec((2, KB_BLK, F), lambda i: (0, i, 0)),
            pl.BlockSpec((KB_BLK, F), lambda i: (i, 0)),
        ],
        out_specs=pl.BlockSpec((KB_BLK, 4), lambda i: (i, 0)),
        out_shape=jax.ShapeDtypeStruct((N, 4), jnp.float32),
    )(v_part, u8)


def kernel(x, v_idx, e_idx, W_msg, W_upd, b_upd):
    v_idx = v_idx.astype(jnp.int32)
    e_idx = e_idx.astype(jnp.int32)
    wm8 = jnp.pad(W_msg, ((0, 0), (0, F - 4)))
    wu8 = jnp.pad(W_upd, ((0, 0), (0, F - 4)))
    c8 = jnp.array([[0, 0, 0, 0, 1, 0, 0, 0]], jnp.float32)
    b8 = jnp.pad(b_upd, (0, F - 4)).reshape(1, F)
    z = jnp.zeros((ZROWS, F), jnp.float32)

    pad = E_PAD - E
    gs1 = jnp.pad(v_idx, (0, pad)).reshape(NGROUPS, GROUP)
    ss1 = jnp.pad(e_idx, (0, pad), constant_values=M).reshape(NGROUPS, GROUP)
    gs2 = jnp.pad(e_idx, (0, pad)).reshape(NGROUPS, GROUP)
    ss2 = jnp.pad(v_idx, (0, pad), constant_values=N).reshape(NGROUPS, GROUP)

    m8, u8 = _ka(x, wm8, wu8, c8, b8)
    e_part = _s1(m8, gs1, ss1, z)
    e_feat8 = _ke(e_part)
    v_part = _s2(e_feat8, gs2, ss2, z)
    return _kb(v_part, u8)


# speedup vs baseline: 4.1081x; 1.6045x over previous
import functools

import jax
import jax.numpy as jnp
from jax import lax
from jax.experimental import pallas as pl
from jax.experimental.pallas import tpu as pltpu
from jax.experimental.pallas import tpu_sc as plsc

N = 10000
M = 5000
E = 320000
D = 128
F = 8

GROUP = 128
NTILES = 32
GP_T = 80
NGROUPS = NTILES * GP_T
E_PAD = NGROUPS * GROUP
NBUF = 8
ROUNDS = GP_T // NBUF
M_PAD = 5120
N_PAD = 10240
ZROWS = N_PAD // 16

ROWS_BLK = 2000
KE_BLK = 640
KB_BLK = 1000


def _sc_stage(acc_rows, table_rows):
    rpt = acc_rows // 16
    tpt = table_rows // 16
    mesh = plsc.VectorSubcoreMesh(core_axis_name="c", subcore_axis_name="s")

    def body(table_hbm, gidx_hbm, sidx_hbm, z_hbm, out_hbm,
             gidxv, sidxv, rows, tbl, acc, *sems):
        gsems = sems[:NBUF]
        ssems = sems[NBUF:]
        cid = lax.axis_index("c")
        sid = lax.axis_index("s")
        wid = cid * 16 + sid
        gbase = wid * GP_T

        pltpu.sync_copy(gidx_hbm.at[pl.ds(gbase, GP_T)], gidxv)
        pltpu.sync_copy(sidx_hbm.at[pl.ds(gbase, GP_T)], sidxv)
        pltpu.sync_copy(table_hbm.at[pl.ds(sid * tpt, tpt)],
                        tbl.at[pl.ds(sid * tpt, tpt)])
        pltpu.sync_copy(z_hbm.at[pl.ds(0, rpt)], acc.at[pl.ds(sid * rpt, rpt)])
        plsc.subcore_barrier()

        for b in range(NBUF):
            pltpu.async_copy(tbl.at[gidxv.at[b]], rows.at[b], gsems[b])

        def round_body(i, carry):
            for b in range(NBUF):
                g = i * NBUF + b
                pb = (b - 1) % NBUF
                pltpu.make_async_copy(
                    tbl.at[gidxv.at[b]], rows.at[b], gsems[b]).wait()
                pltpu.async_copy(rows.at[b], acc.at[sidxv.at[g]], ssems[b],
                                 add=True)

                @pl.when(jnp.logical_and(g >= 1, g <= GP_T - NBUF))
                def _():
                    pltpu.make_async_copy(
                        z_hbm.at[pl.ds(0, GROUP)], rows.at[pb],
                        ssems[pb]).wait()
                    pltpu.async_copy(
                        tbl.at[gidxv.at[g - 1 + NBUF]], rows.at[pb],
                        gsems[pb])
            return carry

        lax.fori_loop(0, ROUNDS, round_body, 0)
        for b in range(NBUF):
            pltpu.make_async_copy(
                z_hbm.at[pl.ds(0, GROUP)], rows.at[b], ssems[b]).wait()
        plsc.subcore_barrier()
        pltpu.sync_copy(acc.at[pl.ds(sid * rpt, rpt)],
                        out_hbm.at[cid, pl.ds(sid * rpt, rpt)])

    return pl.kernel(
        body,
        out_type=jax.ShapeDtypeStruct((2, acc_rows, F), jnp.float32),
        mesh=mesh,
        compiler_params=pltpu.CompilerParams(use_tc_tiling_on_sc=False),
        scratch_types=[
            pltpu.VMEM((GP_T, GROUP), jnp.int32),
            pltpu.VMEM((GP_T, GROUP), jnp.int32),
            pltpu.VMEM((NBUF, GROUP, F), jnp.float32),
            pltpu.VMEM_SHARED((table_rows, F), jnp.float32),
            pltpu.VMEM_SHARED((acc_rows, F), jnp.float32),
        ] + [pltpu.SemaphoreType.DMA] * (2 * NBUF),
    )


_s1 = _sc_stage(M_PAD, N)
_s2 = _sc_stage(N_PAD, M_PAD)


def _ka_body(x_ref, wm_ref, wu_ref, c_ref, b_ref, m8_ref, u8_ref):
    xb = x_ref[...]
    m8_ref[...] = jax.nn.gelu(
        jnp.dot(xb, wm_ref[...], preferred_element_type=jnp.float32)) + c_ref[...]
    u8_ref[...] = jnp.dot(
        xb, wu_ref[...], preferred_element_type=jnp.float32) + b_ref[...]


def _ka(x, wm8, wu8, c8, b8):
    return pl.pallas_call(
        _ka_body,
        grid=(N // ROWS_BLK,),
        in_specs=[
            pl.BlockSpec((ROWS_BLK, D), lambda i: (i, 0)),
            pl.BlockSpec((D, F), lambda i: (0, 0)),
            pl.BlockSpec((D, F), lambda i: (0, 0)),
            pl.BlockSpec((1, F), lambda i: (0, 0)),
            pl.BlockSpec((1, F), lambda i: (0, 0)),
        ],
        out_specs=[
            pl.BlockSpec((ROWS_BLK, F), lambda i: (i, 0)),
            pl.BlockSpec((ROWS_BLK, F), lambda i: (i, 0)),
        ],
        out_shape=[
            jax.ShapeDtypeStruct((N, F), jnp.float32),
            jax.ShapeDtypeStruct((N, F), jnp.float32),
        ],
    )(x, wm8, wu8, c8, b8)


def _ke_body(ep_ref, out_ref):
    p = ep_ref[0] + ep_ref[1]
    cnt = jnp.maximum(p[:, 4:5], 1.0)
    out_ref[...] = p / cnt


def _ke(e_part):
    return pl.pallas_call(
        _ke_body,
        grid=(M_PAD // KE_BLK,),
        in_specs=[pl.BlockSpec((2, KE_BLK, F), lambda i: (0, i, 0))],
        out_specs=pl.BlockSpec((KE_BLK, F), lambda i: (i, 0)),
        out_shape=jax.ShapeDtypeStruct((M_PAD, F), jnp.float32),
    )(e_part)


def _kb_body(vp_ref, u8_ref, out_ref):
    p = vp_ref[0] + vp_ref[1]
    cnt = jnp.maximum(p[:, 4:5], 1.0)
    m_i = p[:, 0:4] / cnt
    h = jax.nn.gelu(u8_ref[:, 0:4] + m_i)
    hm = jnp.max(h, axis=1, keepdims=True)
    out_ref[...] = (h - hm) - jnp.log(
        jnp.sum(jnp.exp(h - hm), axis=1, keepdims=True))


def _kb(v_part, u8):
    return pl.pallas_call(
        _kb_body,
        grid=(N // KB_BLK,),
        in_specs=[
            pl.BlockSpec((2, KB_BLK, F), lambda i: (0, i, 0)),
            pl.BlockSpec((KB_BLK, F), lambda i: (i, 0)),
        ],
        out_specs=pl.BlockSpec((KB_BLK, 4), lambda i: (i, 0)),
        out_shape=jax.ShapeDtypeStruct((N, 4), jnp.float32),
    )(v_part, u8)


def kernel(x, v_idx, e_idx, W_msg, W_upd, b_upd):
    v_idx = v_idx.astype(jnp.int32)
    e_idx = e_idx.astype(jnp.int32)
    wm8 = jnp.pad(W_msg, ((0, 0), (0, F - 4)))
    wu8 = jnp.pad(W_upd, ((0, 0), (0, F - 4)))
    c8 = jnp.array([[0, 0, 0, 0, 1, 0, 0, 0]], jnp.float32)
    b8 = jnp.pad(b_upd, (0, F - 4)).reshape(1, F)
    z = jnp.zeros((ZROWS, F), jnp.float32)

    pad = E_PAD - E
    gs1 = jnp.pad(v_idx, (0, pad)).reshape(NGROUPS, GROUP)
    ss1 = jnp.pad(e_idx, (0, pad), constant_values=M).reshape(NGROUPS, GROUP)
    gs2 = jnp.pad(e_idx, (0, pad)).reshape(NGROUPS, GROUP)
    ss2 = jnp.pad(v_idx, (0, pad), constant_values=N).reshape(NGROUPS, GROUP)

    m8, u8 = _ka(x, wm8, wu8, c8, b8)
    e_part = _s1(m8, gs1, ss1, z)
    e_feat8 = _ke(e_part)
    v_part = _s2(e_feat8, gs2, ss2, z)
    return _kb(v_part, u8)


# speedup vs baseline: 4.6385x; 1.1291x over previous
import functools

import jax
import jax.numpy as jnp
from jax import lax
from jax.experimental import pallas as pl
from jax.experimental.pallas import tpu as pltpu
from jax.experimental.pallas import tpu_sc as plsc

N = 10000
M = 5000
E = 320000
D = 128
F = 8

GROUP = 128
NTILES = 32
NGROUPS = E // GROUP
G_LO = NGROUPS // NTILES
G_REM = NGROUPS % NTILES
GP_T = G_LO + 1
NBUF = 8
ROUNDS = (GP_T + NBUF - 1) // NBUF
M_PAD = 5120
N_PAD = 10240
ZROWS = N_PAD // 16

ROWS_BLK = 2000
KE_BLK = 640
KB_BLK = 1000


def _sc_stage(acc_rows, table_rows):
    rpt = acc_rows // 16
    tpt = table_rows // 16
    mesh = plsc.VectorSubcoreMesh(core_axis_name="c", subcore_axis_name="s")

    def body(table_hbm, gidx_hbm, sidx_hbm, z_hbm, out_hbm,
             gidxv, sidxv, rows, tbl, acc, *sems):
        gsems = sems[:NBUF]
        ssems = sems[NBUF:]
        cid = lax.axis_index("c")
        sid = lax.axis_index("s")
        wid = cid * 16 + sid
        ng = jnp.where(wid < G_REM, G_LO + 1, G_LO)
        gbase = wid * G_LO + jnp.minimum(wid, G_REM)

        pltpu.sync_copy(gidx_hbm.at[pl.ds(gbase, G_LO)],
                        gidxv.at[pl.ds(0, G_LO)])
        pltpu.sync_copy(sidx_hbm.at[pl.ds(gbase, G_LO)],
                        sidxv.at[pl.ds(0, G_LO)])

        @pl.when(wid < G_REM)
        def _():
            pltpu.sync_copy(gidx_hbm.at[pl.ds(gbase + G_LO, 1)],
                            gidxv.at[pl.ds(G_LO, 1)])
            pltpu.sync_copy(sidx_hbm.at[pl.ds(gbase + G_LO, 1)],
                            sidxv.at[pl.ds(G_LO, 1)])
        pltpu.sync_copy(table_hbm.at[pl.ds(sid * tpt, tpt)],
                        tbl.at[pl.ds(sid * tpt, tpt)])
        pltpu.sync_copy(z_hbm.at[pl.ds(0, rpt)], acc.at[pl.ds(sid * rpt, rpt)])
        plsc.subcore_barrier()

        for b in range(NBUF):
            pltpu.async_copy(tbl.at[gidxv.at[b]], rows.at[b], gsems[b])

        def round_body(i, carry):
            for b in range(NBUF):
                g = i * NBUF + b
                pb = (b - 1) % NBUF

                @pl.when(g < ng)
                def _():
                    pltpu.make_async_copy(
                        tbl.at[gidxv.at[b]], rows.at[b], gsems[b]).wait()
                    pltpu.async_copy(rows.at[b], acc.at[sidxv.at[g]],
                                     ssems[b], add=True)

                    @pl.when(jnp.logical_and(g >= 1, g <= ng - NBUF))
                    def _():
                        pltpu.make_async_copy(
                            z_hbm.at[pl.ds(0, GROUP)], rows.at[pb],
                            ssems[pb]).wait()
                        pltpu.async_copy(
                            tbl.at[gidxv.at[g - 1 + NBUF]], rows.at[pb],
                            gsems[pb])
            return carry

        lax.fori_loop(0, ROUNDS, round_body, 0)
        for b in range(NBUF):
            pltpu.make_async_copy(
                z_hbm.at[pl.ds(0, GROUP)], rows.at[b], ssems[b]).wait()
        plsc.subcore_barrier()
        pltpu.sync_copy(acc.at[pl.ds(sid * rpt, rpt)],
                        out_hbm.at[cid, pl.ds(sid * rpt, rpt)])

    return pl.kernel(
        body,
        out_type=jax.ShapeDtypeStruct((2, acc_rows, F), jnp.float32),
        mesh=mesh,
        compiler_params=pltpu.CompilerParams(use_tc_tiling_on_sc=False),
        scratch_types=[
            pltpu.VMEM((GP_T, GROUP), jnp.int32),
            pltpu.VMEM((GP_T, GROUP), jnp.int32),
            pltpu.VMEM((NBUF, GROUP, F), jnp.float32),
            pltpu.VMEM_SHARED((table_rows, F), jnp.float32),
            pltpu.VMEM_SHARED((acc_rows, F), jnp.float32),
        ] + [pltpu.SemaphoreType.DMA] * (2 * NBUF),
    )


_s1 = _sc_stage(M_PAD, N)
_s2 = _sc_stage(N_PAD, M_PAD)


def _ka_body(x_ref, wm_ref, wu_ref, c_ref, b_ref, m8_ref, u8_ref):
    xb = x_ref[...]
    m8_ref[...] = jax.nn.gelu(
        jnp.dot(xb, wm_ref[...], preferred_element_type=jnp.float32)) + c_ref[...]
    u8_ref[...] = jnp.dot(
        xb, wu_ref[...], preferred_element_type=jnp.float32) + b_ref[...]


def _ka(x, wm8, wu8, c8, b8):
    return pl.pallas_call(
        _ka_body,
        grid=(N // ROWS_BLK,),
        in_specs=[
            pl.BlockSpec((ROWS_BLK, D), lambda i: (i, 0)),
            pl.BlockSpec((D, F), lambda i: (0, 0)),
            pl.BlockSpec((D, F), lambda i: (0, 0)),
            pl.BlockSpec((1, F), lambda i: (0, 0)),
            pl.BlockSpec((1, F), lambda i: (0, 0)),
        ],
        out_specs=[
            pl.BlockSpec((ROWS_BLK, F), lambda i: (i, 0)),
            pl.BlockSpec((ROWS_BLK, F), lambda i: (i, 0)),
        ],
        out_shape=[
            jax.ShapeDtypeStruct((N, F), jnp.float32),
            jax.ShapeDtypeStruct((N, F), jnp.float32),
        ],
    )(x, wm8, wu8, c8, b8)


def _ke_body(ep_ref, out_ref):
    p = ep_ref[0] + ep_ref[1]
    cnt = jnp.maximum(p[:, 4:5], 1.0)
    out_ref[...] = p / cnt


def _ke(e_part):
    return pl.pallas_call(
        _ke_body,
        grid=(M_PAD // KE_BLK,),
        in_specs=[pl.BlockSpec((2, KE_BLK, F), lambda i: (0, i, 0))],
        out_specs=pl.BlockSpec((KE_BLK, F), lambda i: (i, 0)),
        out_shape=jax.ShapeDtypeStruct((M_PAD, F), jnp.float32),
    )(e_part)


def _kb_body(vp_ref, u8_ref, out_ref):
    p = vp_ref[0] + vp_ref[1]
    cnt = jnp.maximum(p[:, 4:5], 1.0)
    m_i = p[:, 0:4] / cnt
    h = jax.nn.gelu(u8_ref[:, 0:4] + m_i)
    hm = jnp.max(h, axis=1, keepdims=True)
    out_ref[...] = (h - hm) - jnp.log(
        jnp.sum(jnp.exp(h - hm), axis=1, keepdims=True))


def _kb(v_part, u8):
    return pl.pallas_call(
        _kb_body,
        grid=(N // KB_BLK,),
        in_specs=[
            pl.BlockSpec((2, KB_BLK, F), lambda i: (0, i, 0)),
            pl.BlockSpec((KB_BLK, F), lambda i: (i, 0)),
        ],
        out_specs=pl.BlockSpec((KB_BLK, 4), lambda i: (i, 0)),
        out_shape=jax.ShapeDtypeStruct((N, 4), jnp.float32),
    )(v_part, u8)


def kernel(x, v_idx, e_idx, W_msg, W_upd, b_upd):
    v_idx = v_idx.astype(jnp.int32)
    e_idx = e_idx.astype(jnp.int32)
    wm8 = jnp.pad(W_msg, ((0, 0), (0, F - 4)))
    wu8 = jnp.pad(W_upd, ((0, 0), (0, F - 4)))
    c8 = jnp.array([[0, 0, 0, 0, 1, 0, 0, 0]], jnp.float32)
    b8 = jnp.pad(b_upd, (0, F - 4)).reshape(1, F)
    z = jnp.zeros((ZROWS, F), jnp.float32)

    gs1 = v_idx.reshape(NGROUPS, GROUP)
    ss1 = e_idx.reshape(NGROUPS, GROUP)
    gs2 = ss1
    ss2 = gs1

    m8, u8 = _ka(x, wm8, wu8, c8, b8)
    e_part = _s1(m8, gs1, ss1, z)
    e_feat8 = _ke(e_part)
    v_part = _s2(e_feat8, gs2, ss2, z)
    return _kb(v_part, u8)
